# baseline (speedup 1.0000x reference)
import functools

import jax
import jax.numpy as jnp
from jax import lax
from jax.experimental import pallas as pl
from jax.experimental.pallas import tpu as pltpu
from jax.experimental.pallas import tpu_sc as plsc

B = 16384
F = 26
CARD = 100000
D = 32
FD = F * D
EPS = 1e-5
ROW = CARD + 1

NC = 2
NS = 16
NW = NC * NS
P_PER_W = FD // NW
HALF = B // 2

SPLIT = 50048
RESTB = ROW - SPLIT


def _sc_gather(planes, idxT):
    mesh = plsc.VectorSubcoreMesh(core_axis_name="c", subcore_axis_name="s")

    @functools.partial(
        pl.kernel,
        mesh=mesh,
        compiler_params=pltpu.CompilerParams(use_tc_tiling_on_sc=True,
                                             needs_layout_passes=False),
        out_type=jax.ShapeDtypeStruct((FD, B), jnp.float32),
        scratch_types=[
            pltpu.VMEM((SPLIT,), jnp.float32),
            pltpu.VMEM((RESTB,), jnp.float32),
            pltpu.VMEM((HALF,), jnp.int32),
            pltpu.VMEM((B,), jnp.float32),
            pltpu.SemaphoreType.DMA,
            pltpu.SemaphoreType.DMA,
        ],
    )
    def k(pl_hbm, idx_hbm, out_hbm, buf_a, buf_b, idx_v, out_v, sem_a, sem_b):
        wid = lax.axis_index("s") * NC + lax.axis_index("c")
        p0 = wid * P_PER_W

        pltpu.async_copy(pl_hbm.at[p0, pl.ds(0, SPLIT)], buf_a, sem_a)

        def plane_body(t, _):
            p = p0 + t
            f = p // D
            hb = pltpu.async_copy(pl_hbm.at[p, pl.ds(SPLIT, RESTB)], buf_b,
                                  sem_b)
            pltpu.make_async_copy(pl_hbm.at[p, pl.ds(0, SPLIT)], buf_a,
                                  sem_a).wait()

            def pass_a(h, _):
                pltpu.sync_copy(idx_hbm.at[f, pl.ds(h * HALF, HALF)], idx_v)

                def ga(g, _):
                    i16 = idx_v[pl.ds(g * 16, 16)]
                    ia = jnp.minimum(i16, SPLIT - 1)
                    out_v[pl.ds(h * HALF + g * 16, 16)] = plsc.load_gather(
                        buf_a, [ia])
                    return 0

                lax.fori_loop(0, HALF // 16, ga, 0)
                return 0

            lax.fori_loop(0, 2, pass_a, 0)

            hb.wait()
            pn = p0 + jnp.minimum(t + 1, P_PER_W - 1)
            pltpu.async_copy(pl_hbm.at[pn, pl.ds(0, SPLIT)], buf_a, sem_a)

            def pass_b(h, _):
                pltpu.sync_copy(idx_hbm.at[f, pl.ds(h * HALF, HALF)], idx_v)

                def gb(g, _):
                    s = pl.ds(h * HALF + g * 16, 16)
                    i16 = idx_v[pl.ds(g * 16, 16)]
                    ib = jnp.maximum(i16 - SPLIT, 0)
                    vb = plsc.load_gather(buf_b, [ib])
                    out_v[s] = jnp.where(i16 >= SPLIT, vb, out_v[s])
                    return 0

                lax.fori_loop(0, HALF // 16, gb, 0)
                pltpu.sync_copy(out_v.at[pl.ds(h * HALF, HALF)],
                                out_hbm.at[p, pl.ds(h * HALF, HALF)])
                return 0

            lax.fori_loop(0, 2, pass_b, 0)
            return 0

        lax.fori_loop(0, P_PER_W, plane_body, 0)
        pltpu.make_async_copy(pl_hbm.at[p0, pl.ds(0, SPLIT)], buf_a,
                              sem_a).wait()

    return k(planes, idxT)


BT = 1024


def _tc_body(et_ref, w0, w1, w2, w3, b_ref, g_ref, bt_ref, s_ref, e_ref,
             out_ref):
    hi = jax.lax.Precision.DEFAULT
    dn = (((0,), (0,)), ((), ()))
    et = et_ref[...]
    c0 = lax.dot_general(et[0:256, :], w0[...], dn, precision=hi)
    c1 = lax.dot_general(et[256:512, :], w1[...], dn, precision=hi)
    c2 = lax.dot_general(et[512:768, :], w2[...], dn, precision=hi)
    c3 = lax.dot_general(et[768:832, :], w3[...], dn, precision=hi)
    c = jnp.concatenate([c0, c1, c2, c3], axis=1) + b_ref[...]
    sq = c * c
    msq = jnp.dot(sq, s_ref[...], precision=hi)
    r = lax.rsqrt(msq + EPS)
    scale = jnp.dot(r, e_ref[...], precision=hi)
    out_ref[...] = c * scale * g_ref[...] + bt_ref[...]


def _tc_norm(embT, w0, w1, w2, w3, b832, g832, bt832, S, E):
    grid = (B // BT,)
    full = lambda shape: pl.BlockSpec(shape, lambda i: (0, 0))
    return pl.pallas_call(
        _tc_body,
        grid=grid,
        in_specs=[
            pl.BlockSpec((FD, BT), lambda i: (0, i)),
            full((256, 256)), full((256, 256)), full((256, 256)),
            full((64, 64)),
            full((1, FD)), full((1, FD)), full((1, FD)),
            full((FD, 128)), full((128, FD)),
        ],
        out_specs=pl.BlockSpec((BT, FD), lambda i: (i, 0)),
        out_shape=jax.ShapeDtypeStruct((B, FD), jnp.float32),
    )(embT, w0, w1, w2, w3, b832, g832, bt832, S, E)


def kernel(x, tables, proj_W, proj_b, gamma, beta):
    idxT = jnp.clip(x, 0, CARD).astype(jnp.int32).T
    planes = tables.transpose(0, 2, 1).reshape(FD, ROW)

    C = jnp.eye(D, dtype=jnp.float32) - jnp.full((D, D), 1.0 / D,
                                                 dtype=jnp.float32)
    Wc = jnp.matmul(proj_W, C)
    bc = jnp.matmul(proj_b, C)

    blkdiag = jax.scipy.linalg.block_diag
    w0 = blkdiag(*[Wc[f] for f in range(0, 8)])
    w1 = blkdiag(*[Wc[f] for f in range(8, 16)])
    w2 = blkdiag(*[Wc[f] for f in range(16, 24)])
    w3 = blkdiag(*[Wc[f] for f in range(24, 26)])
    b832 = bc.reshape(1, FD)
    g832 = jnp.tile(gamma, F)[None, :]
    bt832 = jnp.tile(beta, F)[None, :]

    d_ids = jnp.arange(FD, dtype=jnp.int32) // D
    S = (d_ids[:, None] == jnp.arange(128, dtype=jnp.int32)[None, :]
         ).astype(jnp.float32) / D
    E = (jnp.arange(128, dtype=jnp.int32)[:, None] == d_ids[None, :]
         ).astype(jnp.float32)

    embT = _sc_gather(planes, idxT)
    out2 = _tc_norm(embT, w0, w1, w2, w3, b832, g832, bt832, S, E)
    return out2.reshape(B, F, D)


# speedup vs baseline: 1.6406x; 1.6406x over previous
import functools

import jax
import jax.numpy as jnp
from jax import lax
from jax.experimental import pallas as pl
from jax.experimental.pallas import tpu as pltpu
from jax.experimental.pallas import tpu_sc as plsc

B = 16384
F = 26
CARD = 100000
D = 32
FD = F * D
EPS = 1e-5
ROW = CARD + 1

NC = 2
NS = 16
NW = NC * NS
P_PER_W = FD // NW
HALF = B // 2

SPLIT = 50048
RESTB = ROW - SPLIT


def _sc_gather(planes, idxT):
    mesh = plsc.VectorSubcoreMesh(core_axis_name="c", subcore_axis_name="s")

    @functools.partial(
        pl.kernel,
        mesh=mesh,
        compiler_params=pltpu.CompilerParams(use_tc_tiling_on_sc=True,
                                             needs_layout_passes=False),
        out_type=jax.ShapeDtypeStruct((FD, B), jnp.float32),
        scratch_types=[
            pltpu.VMEM((ROW,), jnp.float32),
            pltpu.VMEM((HALF,), jnp.int32),
            pltpu.VMEM((HALF,), jnp.float32),
        ],
    )
    def k(pl_hbm, idx_hbm, out_hbm, plane_v, idx_v, out_v):
        wid = lax.axis_index("s") * NC + lax.axis_index("c")
        p0 = wid * P_PER_W

        def plane_body(t, _):
            p = p0 + t
            f = p // D
            pltpu.sync_copy(pl_hbm.at[p], plane_v)

            def half_body(h, _):
                pltpu.sync_copy(idx_hbm.at[f, pl.ds(h * HALF, HALF)], idx_v)

                def group_body(g, _):
                    i16 = idx_v[pl.ds(g * 16, 16)]
                    out_v[pl.ds(g * 16, 16)] = plsc.load_gather(plane_v, [i16])
                    return 0

                lax.fori_loop(0, HALF // 16, group_body, 0, unroll=8)
                pltpu.sync_copy(out_v, out_hbm.at[p, pl.ds(h * HALF, HALF)])
                return 0

            lax.fori_loop(0, 2, half_body, 0)
            return 0

        lax.fori_loop(0, P_PER_W, plane_body, 0)

    return k(planes, idxT)


BT = 1024


def _tc_body(et_ref, w0, w1, w2, w3, b_ref, g_ref, bt_ref, s_ref, e_ref,
             out_ref):
    hi = jax.lax.Precision.DEFAULT
    dn = (((0,), (0,)), ((), ()))
    et = et_ref[...]
    c0 = lax.dot_general(et[0:256, :], w0[...], dn, precision=hi)
    c1 = lax.dot_general(et[256:512, :], w1[...], dn, precision=hi)
    c2 = lax.dot_general(et[512:768, :], w2[...], dn, precision=hi)
    c3 = lax.dot_general(et[768:832, :], w3[...], dn, precision=hi)
    c = jnp.concatenate([c0, c1, c2, c3], axis=1) + b_ref[...]
    sq = c * c
    msq = jnp.dot(sq, s_ref[...], precision=hi)
    r = lax.rsqrt(msq + EPS)
    scale = jnp.dot(r, e_ref[...], precision=hi)
    out_ref[...] = c * scale * g_ref[...] + bt_ref[...]


def _tc_norm(embT, w0, w1, w2, w3, b832, g832, bt832, S, E):
    grid = (B // BT,)
    full = lambda shape: pl.BlockSpec(shape, lambda i: (0, 0))
    return pl.pallas_call(
        _tc_body,
        grid=grid,
        in_specs=[
            pl.BlockSpec((FD, BT), lambda i: (0, i)),
            full((256, 256)), full((256, 256)), full((256, 256)),
            full((64, 64)),
            full((1, FD)), full((1, FD)), full((1, FD)),
            full((FD, 128)), full((128, FD)),
        ],
        out_specs=pl.BlockSpec((BT, FD), lambda i: (i, 0)),
        out_shape=jax.ShapeDtypeStruct((B, FD), jnp.float32),
    )(embT, w0, w1, w2, w3, b832, g832, bt832, S, E)


def kernel(x, tables, proj_W, proj_b, gamma, beta):
    idxT = jnp.clip(x, 0, CARD).astype(jnp.int32).T
    planes = tables.transpose(0, 2, 1).reshape(FD, ROW)

    C = jnp.eye(D, dtype=jnp.float32) - jnp.full((D, D), 1.0 / D,
                                                 dtype=jnp.float32)
    Wc = jnp.matmul(proj_W, C)
    bc = jnp.matmul(proj_b, C)

    blkdiag = jax.scipy.linalg.block_diag
    w0 = blkdiag(*[Wc[f] for f in range(0, 8)])
    w1 = blkdiag(*[Wc[f] for f in range(8, 16)])
    w2 = blkdiag(*[Wc[f] for f in range(16, 24)])
    w3 = blkdiag(*[Wc[f] for f in range(24, 26)])
    b832 = bc.reshape(1, FD)
    g832 = jnp.tile(gamma, F)[None, :]
    bt832 = jnp.tile(beta, F)[None, :]

    d_ids = jnp.arange(FD, dtype=jnp.int32) // D
    S = (d_ids[:, None] == jnp.arange(128, dtype=jnp.int32)[None, :]
         ).astype(jnp.float32) / D
    E = (jnp.arange(128, dtype=jnp.int32)[:, None] == d_ids[None, :]
         ).astype(jnp.float32)

    embT = _sc_gather(planes, idxT)
    out2 = _tc_norm(embT, w0, w1, w2, w3, b832, g832, bt832, S, E)
    return out2.reshape(B, F, D)


# speedup vs baseline: 1.7758x; 1.0824x over previous
import functools

import jax
import jax.numpy as jnp
from jax import lax
from jax.experimental import pallas as pl
from jax.experimental.pallas import tpu as pltpu
from jax.experimental.pallas import tpu_sc as plsc

B = 16384
F = 26
CARD = 100000
D = 32
FD = F * D
EPS = 1e-5
ROW = CARD + 1

NC = 2
NS = 16
NW = NC * NS
P_PER_W = FD // NW
HALF = B // 2

SPLIT = 50048
RESTB = ROW - SPLIT


def _sc_gather(planes, idxT):
    mesh = plsc.VectorSubcoreMesh(core_axis_name="c", subcore_axis_name="s")

    @functools.partial(
        pl.kernel,
        mesh=mesh,
        compiler_params=pltpu.CompilerParams(use_tc_tiling_on_sc=True,
                                             needs_layout_passes=False),
        out_type=jax.ShapeDtypeStruct((FD, B), jnp.float32),
        scratch_types=[
            pltpu.VMEM((ROW,), jnp.float32),
            pltpu.VMEM((B,), jnp.int32),
            pltpu.VMEM((HALF,), jnp.float32),
        ],
    )
    def k(pl_hbm, idx_hbm, out_hbm, plane_v, idx_v, out_v):
        wid = lax.axis_index("s") * NC + lax.axis_index("c")
        p0 = wid * P_PER_W

        def plane_body(t, f_prev):
            p = p0 + t
            f = p // D

            @pl.when(f != f_prev)
            def _():
                pltpu.sync_copy(idx_hbm.at[f], idx_v)

            pltpu.sync_copy(pl_hbm.at[p], plane_v)

            def half_body(h, _):
                def group_body(g, _):
                    i16 = idx_v[pl.ds(h * HALF + g * 16, 16)]
                    out_v[pl.ds(g * 16, 16)] = plsc.load_gather(plane_v, [i16])
                    return 0

                lax.fori_loop(0, HALF // 16, group_body, 0)
                pltpu.sync_copy(out_v, out_hbm.at[p, pl.ds(h * HALF, HALF)])
                return 0

            lax.fori_loop(0, 2, half_body, 0)
            return f

        lax.fori_loop(0, P_PER_W, plane_body, jnp.int32(-1))

    return k(planes, idxT)


BT = 1024


def _tc_body(et_ref, w0, w1, w2, w3, b_ref, g_ref, bt_ref, s_ref, e_ref,
             out_ref):
    hi = jax.lax.Precision.DEFAULT
    dn = (((0,), (0,)), ((), ()))
    et = et_ref[...]
    c0 = lax.dot_general(et[0:256, :], w0[...], dn, precision=hi)
    c1 = lax.dot_general(et[256:512, :], w1[...], dn, precision=hi)
    c2 = lax.dot_general(et[512:768, :], w2[...], dn, precision=hi)
    c3 = lax.dot_general(et[768:832, :], w3[...], dn, precision=hi)
    c = jnp.concatenate([c0, c1, c2, c3], axis=1) + b_ref[...]
    sq = c * c
    msq = jnp.dot(sq, s_ref[...], precision=hi)
    r = lax.rsqrt(msq + EPS)
    scale = jnp.dot(r, e_ref[...], precision=hi)
    out_ref[...] = c * scale * g_ref[...] + bt_ref[...]


def _tc_norm(embT, w0, w1, w2, w3, b832, g832, bt832, S, E):
    grid = (B // BT,)
    full = lambda shape: pl.BlockSpec(shape, lambda i: (0, 0))
    return pl.pallas_call(
        _tc_body,
        grid=grid,
        in_specs=[
            pl.BlockSpec((FD, BT), lambda i: (0, i)),
            full((256, 256)), full((256, 256)), full((256, 256)),
            full((64, 64)),
            full((1, FD)), full((1, FD)), full((1, FD)),
            full((FD, 128)), full((128, FD)),
        ],
        out_specs=pl.BlockSpec((BT, FD), lambda i: (i, 0)),
        out_shape=jax.ShapeDtypeStruct((B, FD), jnp.float32),
    )(embT, w0, w1, w2, w3, b832, g832, bt832, S, E)


def kernel(x, tables, proj_W, proj_b, gamma, beta):
    idxT = jnp.clip(x, 0, CARD).astype(jnp.int32).T
    planes = tables.transpose(0, 2, 1).reshape(FD, ROW)

    C = jnp.eye(D, dtype=jnp.float32) - jnp.full((D, D), 1.0 / D,
                                                 dtype=jnp.float32)
    Wc = jnp.matmul(proj_W, C)
    bc = jnp.matmul(proj_b, C)

    blkdiag = jax.scipy.linalg.block_diag
    w0 = blkdiag(*[Wc[f] for f in range(0, 8)])
    w1 = blkdiag(*[Wc[f] for f in range(8, 16)])
    w2 = blkdiag(*[Wc[f] for f in range(16, 24)])
    w3 = blkdiag(*[Wc[f] for f in range(24, 26)])
    b832 = bc.reshape(1, FD)
    g832 = jnp.tile(gamma, F)[None, :]
    bt832 = jnp.tile(beta, F)[None, :]

    d_ids = jnp.arange(FD, dtype=jnp.int32) // D
    S = (d_ids[:, None] == jnp.arange(128, dtype=jnp.int32)[None, :]
         ).astype(jnp.float32) / D
    E = (jnp.arange(128, dtype=jnp.int32)[:, None] == d_ids[None, :]
         ).astype(jnp.float32)

    embT = _sc_gather(planes, idxT)
    out2 = _tc_norm(embT, w0, w1, w2, w3, b832, g832, bt832, S, E)
    return out2.reshape(B, F, D)


# speedup vs baseline: 1.8297x; 1.0303x over previous
import functools

import jax
import jax.numpy as jnp
from jax import lax
from jax.experimental import pallas as pl
from jax.experimental.pallas import tpu as pltpu
from jax.experimental.pallas import tpu_sc as plsc

B = 16384
F = 26
CARD = 100000
D = 32
FD = F * D
EPS = 1e-5
ROW = CARD + 1

NC = 2
NS = 16
NW = NC * NS
P_PER_W = FD // NW
HALF = B // 2

SPLIT = 50048
RESTB = ROW - SPLIT


def _sc_gather(planes, idxT):
    mesh = plsc.VectorSubcoreMesh(core_axis_name="c", subcore_axis_name="s")

    @functools.partial(
        pl.kernel,
        mesh=mesh,
        compiler_params=pltpu.CompilerParams(use_tc_tiling_on_sc=True,
                                             needs_layout_passes=False),
        out_type=jax.ShapeDtypeStruct((FD, B), jnp.float32),
        scratch_types=[
            pltpu.VMEM((ROW,), jnp.float32),
            pltpu.VMEM((HALF,), jnp.int32),
            pltpu.VMEM((HALF,), jnp.float32),
        ],
    )
    def k(pl_hbm, idx_hbm, out_hbm, plane_v, idx_v, out_v):
        wid = lax.axis_index("s") * NC + lax.axis_index("c")
        p0 = wid * P_PER_W

        def plane_body(t, _):
            p = p0 + t
            f = p // D
            pltpu.sync_copy(pl_hbm.at[p], plane_v)

            def half_body(h, _):
                pltpu.sync_copy(idx_hbm.at[f, pl.ds(h * HALF, HALF)], idx_v)

                def group_body(g, _):
                    i16 = idx_v[pl.ds(g * 16, 16)]
                    out_v[pl.ds(g * 16, 16)] = plsc.load_gather(plane_v, [i16])
                    return 0

                lax.fori_loop(0, HALF // 16, group_body, 0)
                pltpu.sync_copy(out_v, out_hbm.at[p, pl.ds(h * HALF, HALF)])
                return 0

            lax.fori_loop(0, 2, half_body, 0)
            return 0

        lax.fori_loop(0, P_PER_W, plane_body, 0)

    return k(planes, idxT)


BT = 1024
BTX = 2048


def _idx_t_body(x_ref, out_ref):
    xb = jnp.clip(x_ref[...], 0, CARD).astype(jnp.int32)
    out_ref[...] = xb.T


def _idx_transpose(x):
    return pl.pallas_call(
        _idx_t_body,
        grid=(B // BTX,),
        in_specs=[pl.BlockSpec((BTX, F), lambda i: (i, 0))],
        out_specs=pl.BlockSpec((F, BTX), lambda i: (0, i)),
        out_shape=jax.ShapeDtypeStruct((F, B), jnp.int32),
    )(x)


def _tc_body(et_ref, w0, w1, w2, w3, b_ref, g_ref, bt_ref, s_ref, e_ref,
             out_ref):
    hi = jax.lax.Precision.DEFAULT
    dn = (((0,), (0,)), ((), ()))
    et = et_ref[...]
    c0 = lax.dot_general(et[0:256, :], w0[...], dn, precision=hi)
    c1 = lax.dot_general(et[256:512, :], w1[...], dn, precision=hi)
    c2 = lax.dot_general(et[512:768, :], w2[...], dn, precision=hi)
    c3 = lax.dot_general(et[768:832, :], w3[...], dn, precision=hi)
    c = jnp.concatenate([c0, c1, c2, c3], axis=1) + b_ref[...]
    sq = c * c
    msq = jnp.dot(sq, s_ref[...], precision=hi)
    r = lax.rsqrt(msq + EPS)
    scale = jnp.dot(r, e_ref[...], precision=hi)
    out_ref[...] = c * scale * g_ref[...] + bt_ref[...]


def _tc_norm(embT, w0, w1, w2, w3, b832, g832, bt832, S, E):
    grid = (B // BT,)
    full = lambda shape: pl.BlockSpec(shape, lambda i: (0, 0))
    return pl.pallas_call(
        _tc_body,
        grid=grid,
        in_specs=[
            pl.BlockSpec((FD, BT), lambda i: (0, i)),
            full((256, 256)), full((256, 256)), full((256, 256)),
            full((64, 64)),
            full((1, FD)), full((1, FD)), full((1, FD)),
            full((FD, 128)), full((128, FD)),
        ],
        out_specs=pl.BlockSpec((BT, FD), lambda i: (i, 0)),
        out_shape=jax.ShapeDtypeStruct((B, FD), jnp.float32),
    )(embT, w0, w1, w2, w3, b832, g832, bt832, S, E)


def kernel(x, tables, proj_W, proj_b, gamma, beta):
    idxT = _idx_transpose(x)
    planes = tables.transpose(0, 2, 1).reshape(FD, ROW)

    C = jnp.eye(D, dtype=jnp.float32) - jnp.full((D, D), 1.0 / D,
                                                 dtype=jnp.float32)
    Wc = jnp.matmul(proj_W, C)
    bc = jnp.matmul(proj_b, C)

    blkdiag = jax.scipy.linalg.block_diag
    w0 = blkdiag(*[Wc[f] for f in range(0, 8)])
    w1 = blkdiag(*[Wc[f] for f in range(8, 16)])
    w2 = blkdiag(*[Wc[f] for f in range(16, 24)])
    w3 = blkdiag(*[Wc[f] for f in range(24, 26)])
    b832 = bc.reshape(1, FD)
    g832 = jnp.tile(gamma, F)[None, :]
    bt832 = jnp.tile(beta, F)[None, :]

    d_ids = jnp.arange(FD, dtype=jnp.int32) // D
    S = (d_ids[:, None] == jnp.arange(128, dtype=jnp.int32)[None, :]
         ).astype(jnp.float32) / D
    E = (jnp.arange(128, dtype=jnp.int32)[:, None] == d_ids[None, :]
         ).astype(jnp.float32)

    embT = _sc_gather(planes, idxT)
    out2 = _tc_norm(embT, w0, w1, w2, w3, b832, g832, bt832, S, E)
    return out2.reshape(B, F, D)


# speedup vs baseline: 1.9367x; 1.0585x over previous
import functools

import jax
import jax.numpy as jnp
from jax import lax
from jax.experimental import pallas as pl
from jax.experimental.pallas import tpu as pltpu
from jax.experimental.pallas import tpu_sc as plsc

B = 16384
F = 26
CARD = 100000
D = 32
FD = F * D
EPS = 1e-5
ROW = CARD + 1

NC = 2
NS = 16
NW = NC * NS
P_PER_W = FD // NW
HALF = B // 2

SPLIT = 50048
RESTB = ROW - SPLIT


def _sc_gather(planes, idxT):
    mesh = plsc.VectorSubcoreMesh(core_axis_name="c", subcore_axis_name="s")

    @functools.partial(
        pl.kernel,
        mesh=mesh,
        compiler_params=pltpu.CompilerParams(use_tc_tiling_on_sc=True,
                                             needs_layout_passes=False),
        out_type=jax.ShapeDtypeStruct((FD, B), jnp.float32),
        scratch_types=[
            pltpu.VMEM((ROW,), jnp.float32),
            pltpu.VMEM((HALF,), jnp.int32),
            pltpu.VMEM((HALF,), jnp.float32),
            pltpu.SemaphoreType.DMA,
            pltpu.SemaphoreType.DMA,
        ],
    )
    def k(pl_hbm, idx_hbm, out_hbm, plane_v, idx_v, out_v, sem_a, sem_b):
        wid = lax.axis_index("s") * NC + lax.axis_index("c")
        p0 = wid * P_PER_W

        def plane_body(t, _):
            p = p0 + t
            f = p // D
            ha = pltpu.async_copy(pl_hbm.at[p], plane_v, sem_a)
            pltpu.sync_copy(idx_hbm.at[f, pl.ds(0, HALF)], idx_v)
            ha.wait()

            def half_body(h, _):
                @pl.when(h == 1)
                def _():
                    pltpu.sync_copy(idx_hbm.at[f, pl.ds(HALF, HALF)], idx_v)

                def group_body(g, _):
                    i16 = idx_v[pl.ds(g * 16, 16)]
                    out_v[pl.ds(g * 16, 16)] = plsc.load_gather(plane_v, [i16])
                    return 0

                lax.fori_loop(0, HALF // 16, group_body, 0)
                pltpu.sync_copy(out_v, out_hbm.at[p, pl.ds(h * HALF, HALF)])
                return 0

            lax.fori_loop(0, 2, half_body, 0)
            return 0

        lax.fori_loop(0, P_PER_W, plane_body, 0)

    return k(planes, idxT)


BT = 1024


def _tc_body(et_ref, w0, w1, w2, w3, b_ref, g_ref, bt_ref, s_ref, e_ref,
             out_ref):
    hi = jax.lax.Precision.DEFAULT
    dn = (((0,), (0,)), ((), ()))
    et = et_ref[...]
    c0 = lax.dot_general(et[0:256, :], w0[...], dn, precision=hi)
    c1 = lax.dot_general(et[256:512, :], w1[...], dn, precision=hi)
    c2 = lax.dot_general(et[512:768, :], w2[...], dn, precision=hi)
    c3 = lax.dot_general(et[768:832, :], w3[...], dn, precision=hi)
    c = jnp.concatenate([c0, c1, c2, c3], axis=1) + b_ref[...]
    sq = c * c
    msq = jnp.dot(sq, s_ref[...], precision=hi)
    r = lax.rsqrt(msq + EPS)
    scale = jnp.dot(r, e_ref[...], precision=hi)
    out_ref[...] = c * scale * g_ref[...] + bt_ref[...]


def _tc_norm(embT, w0, w1, w2, w3, b832, g832, bt832, S, E):
    grid = (B // BT,)
    full = lambda shape: pl.BlockSpec(shape, lambda i: (0, 0))
    return pl.pallas_call(
        _tc_body,
        grid=grid,
        in_specs=[
            pl.BlockSpec((FD, BT), lambda i: (0, i)),
            full((256, 256)), full((256, 256)), full((256, 256)),
            full((64, 64)),
            full((1, FD)), full((1, FD)), full((1, FD)),
            full((FD, 128)), full((128, FD)),
        ],
        out_specs=pl.BlockSpec((BT, FD), lambda i: (i, 0)),
        out_shape=jax.ShapeDtypeStruct((B, FD), jnp.float32),
    )(embT, w0, w1, w2, w3, b832, g832, bt832, S, E)


def kernel(x, tables, proj_W, proj_b, gamma, beta):
    idxT = jnp.clip(x, 0, CARD).astype(jnp.int32).T
    planes = tables.transpose(0, 2, 1).reshape(FD, ROW)

    C = jnp.eye(D, dtype=jnp.float32) - jnp.full((D, D), 1.0 / D,
                                                 dtype=jnp.float32)
    Wc = jnp.matmul(proj_W, C)
    bc = jnp.matmul(proj_b, C)

    blkdiag = jax.scipy.linalg.block_diag
    w0 = blkdiag(*[Wc[f] for f in range(0, 8)])
    w1 = blkdiag(*[Wc[f] for f in range(8, 16)])
    w2 = blkdiag(*[Wc[f] for f in range(16, 24)])
    w3 = blkdiag(*[Wc[f] for f in range(24, 26)])
    b832 = bc.reshape(1, FD)
    g832 = jnp.tile(gamma, F)[None, :]
    bt832 = jnp.tile(beta, F)[None, :]

    d_ids = jnp.arange(FD, dtype=jnp.int32) // D
    S = (d_ids[:, None] == jnp.arange(128, dtype=jnp.int32)[None, :]
         ).astype(jnp.float32) / D
    E = (jnp.arange(128, dtype=jnp.int32)[:, None] == d_ids[None, :]
         ).astype(jnp.float32)

    embT = _sc_gather(planes, idxT)
    out2 = _tc_norm(embT, w0, w1, w2, w3, b832, g832, bt832, S, E)
    return out2.reshape(B, F, D)


# speedup vs baseline: 1.9871x; 1.0260x over previous
import functools

import jax
import jax.numpy as jnp
from jax import lax
from jax.experimental import pallas as pl
from jax.experimental.pallas import tpu as pltpu
from jax.experimental.pallas import tpu_sc as plsc

B = 16384
F = 26
CARD = 100000
D = 32
FD = F * D
EPS = 1e-5
ROW = CARD + 1

NC = 2
NS = 16
NW = NC * NS
P_PER_W = FD // NW
HALF = B // 2

SPLIT = 50048
RESTB = ROW - SPLIT


def _sc_gather(planes, idxT):
    mesh = plsc.VectorSubcoreMesh(core_axis_name="c", subcore_axis_name="s")

    @functools.partial(
        pl.kernel,
        mesh=mesh,
        compiler_params=pltpu.CompilerParams(use_tc_tiling_on_sc=True,
                                             needs_layout_passes=False),
        out_type=jax.ShapeDtypeStruct((FD, B), jnp.float32),
        scratch_types=[
            pltpu.VMEM((ROW,), jnp.float32),
            pltpu.VMEM((HALF,), jnp.int32),
            pltpu.VMEM((HALF,), jnp.int32),
            pltpu.VMEM((HALF,), jnp.float32),
            pltpu.SemaphoreType.DMA,
            pltpu.SemaphoreType.DMA,
        ],
    )
    def k(pl_hbm, idx_hbm, out_hbm, plane_v, idx0_v, idx1_v, out_v, sem_a,
          sem_b):
        wid = lax.axis_index("s") * NC + lax.axis_index("c")
        p0 = wid * P_PER_W

        def plane_body(t, _):
            p = p0 + t
            f = p // D
            ha = pltpu.async_copy(pl_hbm.at[p], plane_v, sem_a)
            hb = pltpu.async_copy(idx_hbm.at[f, pl.ds(HALF, HALF)], idx1_v,
                                  sem_b)
            pltpu.sync_copy(idx_hbm.at[f, pl.ds(0, HALF)], idx0_v)
            hb.wait()
            ha.wait()

            for h, idx_v in ((0, idx0_v), (1, idx1_v)):
                def group_body(g, _, idx_v=idx_v):
                    i16 = idx_v[pl.ds(g * 16, 16)]
                    out_v[pl.ds(g * 16, 16)] = plsc.load_gather(plane_v, [i16])
                    return 0

                lax.fori_loop(0, HALF // 16, group_body, 0)
                pltpu.sync_copy(out_v, out_hbm.at[p, pl.ds(h * HALF, HALF)])
            return 0

        lax.fori_loop(0, P_PER_W, plane_body, 0)

    return k(planes, idxT)


BT = 1024


def _tc_body(et_ref, w0, w1, w2, w3, b_ref, g_ref, bt_ref, s_ref, e_ref,
             out_ref):
    hi = jax.lax.Precision.DEFAULT
    dn = (((0,), (0,)), ((), ()))
    et = et_ref[...]
    c0 = lax.dot_general(et[0:256, :], w0[...], dn, precision=hi)
    c1 = lax.dot_general(et[256:512, :], w1[...], dn, precision=hi)
    c2 = lax.dot_general(et[512:768, :], w2[...], dn, precision=hi)
    c3 = lax.dot_general(et[768:832, :], w3[...], dn, precision=hi)
    c = jnp.concatenate([c0, c1, c2, c3], axis=1) + b_ref[...]
    sq = c * c
    msq = jnp.dot(sq, s_ref[...], precision=hi)
    r = lax.rsqrt(msq + EPS)
    scale = jnp.dot(r, e_ref[...], precision=hi)
    out_ref[...] = c * scale * g_ref[...] + bt_ref[...]


def _tc_norm(embT, w0, w1, w2, w3, b832, g832, bt832, S, E):
    grid = (B // BT,)
    full = lambda shape: pl.BlockSpec(shape, lambda i: (0, 0))
    return pl.pallas_call(
        _tc_body,
        grid=grid,
        in_specs=[
            pl.BlockSpec((FD, BT), lambda i: (0, i)),
            full((256, 256)), full((256, 256)), full((256, 256)),
            full((64, 64)),
            full((1, FD)), full((1, FD)), full((1, FD)),
            full((FD, 128)), full((128, FD)),
        ],
        out_specs=pl.BlockSpec((BT, FD), lambda i: (i, 0)),
        out_shape=jax.ShapeDtypeStruct((B, FD), jnp.float32),
    )(embT, w0, w1, w2, w3, b832, g832, bt832, S, E)


def kernel(x, tables, proj_W, proj_b, gamma, beta):
    idxT = jnp.clip(x, 0, CARD).astype(jnp.int32).T
    planes = tables.transpose(0, 2, 1).reshape(FD, ROW)

    C = jnp.eye(D, dtype=jnp.float32) - jnp.full((D, D), 1.0 / D,
                                                 dtype=jnp.float32)
    Wc = jnp.matmul(proj_W, C)
    bc = jnp.matmul(proj_b, C)

    blkdiag = jax.scipy.linalg.block_diag
    w0 = blkdiag(*[Wc[f] for f in range(0, 8)])
    w1 = blkdiag(*[Wc[f] for f in range(8, 16)])
    w2 = blkdiag(*[Wc[f] for f in range(16, 24)])
    w3 = blkdiag(*[Wc[f] for f in range(24, 26)])
    b832 = bc.reshape(1, FD)
    g832 = jnp.tile(gamma, F)[None, :]
    bt832 = jnp.tile(beta, F)[None, :]

    d_ids = jnp.arange(FD, dtype=jnp.int32) // D
    S = (d_ids[:, None] == jnp.arange(128, dtype=jnp.int32)[None, :]
         ).astype(jnp.float32) / D
    E = (jnp.arange(128, dtype=jnp.int32)[:, None] == d_ids[None, :]
         ).astype(jnp.float32)

    embT = _sc_gather(planes, idxT)
    out2 = _tc_norm(embT, w0, w1, w2, w3, b832, g832, bt832, S, E)
    return out2.reshape(B, F, D)
